# Initial kernel scaffold; baseline (speedup 1.0000x reference)
import jax
import jax.numpy as jnp
from jax.experimental import pallas as pl


def kernel(z, codes):
    raise NotImplementedError("write your pallas kernel here")


# speedup vs baseline: 7.4585x; 7.4585x over previous
import functools

import jax
import jax.numpy as jnp
from jax.experimental import pallas as pl

_NUM_CODES = 512
_LATENT = 64


def _vq_tile(x_ref, w_ref, soft_ref, hard_ref, idx_ref):
    x = x_ref[0, 0, :]
    w = w_ref[0, 0, :]
    neg = -jnp.abs(x[:, None] - w[None, :])
    m = jnp.max(neg, axis=1, keepdims=True)
    e = jnp.exp(neg - m)
    denom = jnp.sum(e, axis=1)
    num = jnp.sum(e * w[None, :], axis=1)
    soft_ref[0, 0, :] = num / denom
    iota = jax.lax.broadcasted_iota(jnp.int32, neg.shape, 1)
    idx = jnp.min(jnp.where(neg == m, iota, jnp.int32(_NUM_CODES)), axis=1)
    idx_ref[0, 0, :] = idx
    hard_ref[0, 0, :] = jnp.sum(
        jnp.where(iota == idx[:, None], w[None, :], 0.0), axis=1
    )


@functools.partial(jax.jit, static_argnames=("interpret",))
def _run(z, codes, interpret=False):
    B, C, H, W = z.shape
    K = codes.shape[1]
    S = B * H * W
    SBLK = 768
    xs = z.reshape(B, C, H * W).transpose(1, 0, 2).reshape(C, 1, S)
    w3 = codes.reshape(C, 1, K)
    out_shape = [
        jax.ShapeDtypeStruct((C, 1, S), jnp.float32),
        jax.ShapeDtypeStruct((C, 1, S), jnp.float32),
        jax.ShapeDtypeStruct((C, 1, S), jnp.int32),
    ]
    grid = (C, S // SBLK)
    soft, hard, idx = pl.pallas_call(
        _vq_tile,
        grid=grid,
        in_specs=[
            pl.BlockSpec((1, 1, SBLK), lambda c, s: (c, 0, s)),
            pl.BlockSpec((1, 1, K), lambda c, s: (c, 0, 0)),
        ],
        out_specs=[
            pl.BlockSpec((1, 1, SBLK), lambda c, s: (c, 0, s)),
            pl.BlockSpec((1, 1, SBLK), lambda c, s: (c, 0, s)),
            pl.BlockSpec((1, 1, SBLK), lambda c, s: (c, 0, s)),
        ],
        out_shape=out_shape,
        interpret=interpret,
    )(xs, w3)

    def back(a):
        return a.reshape(C, B, H, W).transpose(1, 2, 3, 0)

    return back(soft), back(hard), back(idx)


def kernel(z, codes):
    return _run(z, codes)


# speedup vs baseline: 14.1751x; 1.9005x over previous
import functools

import jax
import jax.numpy as jnp
from jax.experimental import pallas as pl

_NUM_CODES = 512
_LATENT = 64


_NEG_LOG2E = -1.4426950408889634


def _vq_tile(x_ref, w_ref, soft_ref, hard_ref, idx_ref):
    x = x_ref[0, 0, :]
    w = w_ref[0]
    t = w - x[None, :]
    d = jnp.abs(t)
    dmin = jnp.min(d, axis=0, keepdims=True)
    e = jnp.exp2(d * jnp.float32(_NEG_LOG2E))
    denom = jnp.sum(e, axis=0)
    num_t = jnp.sum(e * t, axis=0)
    soft_ref[0, 0, :] = x + num_t / denom
    iota2 = jax.lax.broadcasted_iota(jnp.int32, d.shape, 0) * 2
    key = iota2 + (t < 0).astype(jnp.int32)
    packed = jnp.min(
        jnp.where(d == dmin, key, jnp.int32(2 * _NUM_CODES)), axis=0
    )
    idx_ref[0, 0, :] = packed >> 1
    sign = 1.0 - 2.0 * (packed & 1).astype(jnp.float32)
    hard_ref[0, 0, :] = x + sign * dmin[0, :]


@functools.partial(jax.jit, static_argnames=("interpret",))
def _run(z, codes, interpret=False):
    B, C, H, W = z.shape
    K = codes.shape[1]
    S = B * H * W
    SBLK = 768
    xs = z.reshape(B, C, H * W).transpose(1, 0, 2).reshape(C, 1, S)
    w3 = codes.reshape(C, K, 1)
    out_shape = [
        jax.ShapeDtypeStruct((C, 1, S), jnp.float32),
        jax.ShapeDtypeStruct((C, 1, S), jnp.float32),
        jax.ShapeDtypeStruct((C, 1, S), jnp.int32),
    ]
    grid = (C, S // SBLK)
    soft, hard, idx = pl.pallas_call(
        _vq_tile,
        grid=grid,
        in_specs=[
            pl.BlockSpec((1, 1, SBLK), lambda c, s: (c, 0, s)),
            pl.BlockSpec((1, K, 1), lambda c, s: (c, 0, 0)),
        ],
        out_specs=[
            pl.BlockSpec((1, 1, SBLK), lambda c, s: (c, 0, s)),
            pl.BlockSpec((1, 1, SBLK), lambda c, s: (c, 0, s)),
            pl.BlockSpec((1, 1, SBLK), lambda c, s: (c, 0, s)),
        ],
        out_shape=out_shape,
        interpret=interpret,
    )(xs, w3)

    def back(a):
        return a.reshape(C, B, H, W).transpose(1, 2, 3, 0)

    return back(soft), back(hard), back(idx)


def kernel(z, codes):
    return _run(z, codes)


# speedup vs baseline: 18.5059x; 1.3055x over previous
import functools

import jax
import jax.numpy as jnp
from jax.experimental import pallas as pl
from jax.experimental.pallas import tpu as pltpu

_NUM_CODES = 512
_LATENT = 64
_NEG_LOG2E = -1.4426950408889634


def _vq_tile(x_ref, w_ref, lhs_ref, soft_ref, hard_ref, idx_ref):
    x = x_ref[0, 0, :]
    w = w_ref[0]
    t = w - x[None, :]
    d = jnp.abs(t)
    dmin = jnp.min(d, axis=0, keepdims=True)
    e = jnp.exp2(d * jnp.float32(_NEG_LOG2E))
    nd = jax.lax.dot_general(
        lhs_ref[0], e, (((1,), (0,)), ((), ())),
        preferred_element_type=jnp.float32,
    )
    soft_ref[0, 0, :] = nd[0, :] / nd[1, :]
    iota2 = jax.lax.broadcasted_iota(jnp.int32, d.shape, 0) * 2
    key = iota2 + (t < 0).astype(jnp.int32)
    packed = jnp.min(
        jnp.where(d == dmin, key, jnp.int32(2 * _NUM_CODES)), axis=0
    )
    idx_ref[0, 0, :] = packed >> 1
    sign = 1.0 - 2.0 * (packed & 1).astype(jnp.float32)
    hard_ref[0, 0, :] = x + sign * dmin[0, :]


@functools.partial(jax.jit, static_argnames=("interpret",))
def _run(z, codes, interpret=False):
    B, C, H, W = z.shape
    K = codes.shape[1]
    S = B * H * W
    SBLK = 2304
    xs = z.reshape(B, C, H * W).transpose(1, 0, 2).reshape(C, 1, S)
    wcol = codes.reshape(C, K, 1)
    lhs = jnp.stack([codes, jnp.ones_like(codes)], axis=1)
    out_shape = [
        jax.ShapeDtypeStruct((C, 1, S), jnp.float32),
        jax.ShapeDtypeStruct((C, 1, S), jnp.float32),
        jax.ShapeDtypeStruct((C, 1, S), jnp.int32),
    ]
    grid = (C, S // SBLK)
    soft, hard, idx = pl.pallas_call(
        _vq_tile,
        grid=grid,
        in_specs=[
            pl.BlockSpec((1, 1, SBLK), lambda c, s: (c, 0, s)),
            pl.BlockSpec((1, K, 1), lambda c, s: (c, 0, 0)),
            pl.BlockSpec((1, 2, K), lambda c, s: (c, 0, 0)),
        ],
        out_specs=[
            pl.BlockSpec((1, 1, SBLK), lambda c, s: (c, 0, s)),
            pl.BlockSpec((1, 1, SBLK), lambda c, s: (c, 0, s)),
            pl.BlockSpec((1, 1, SBLK), lambda c, s: (c, 0, s)),
        ],
        out_shape=out_shape,
        compiler_params=pltpu.CompilerParams(
            dimension_semantics=("parallel", "parallel"),
        ),
        interpret=interpret,
    )(xs, wcol, lhs)

    def back(a):
        return a.reshape(C, B, H, W).transpose(1, 2, 3, 0)

    return back(soft), back(hard), back(idx)


def kernel(z, codes):
    return _run(z, codes)


# speedup vs baseline: 20.4988x; 1.1077x over previous
import functools

import jax
import jax.numpy as jnp
from jax.experimental import pallas as pl
from jax.experimental.pallas import tpu as pltpu

_NUM_CODES = 512
_LATENT = 64
_NEG_LOG2E = -1.4426950408889634


def _vq_tile(x_ref, w_ref, lhs_ref, base_ref, soft_ref, hard_ref, idx_ref):
    x = x_ref[0, 0, :]
    w = w_ref[0]
    t = w - x[None, :]
    d = jnp.abs(t)
    dmin = jnp.min(d, axis=0, keepdims=True)
    e = jnp.exp2(d * jnp.float32(_NEG_LOG2E))
    nd = jax.lax.dot_general(
        lhs_ref[0], e, (((1,), (0,)), ((), ())),
        preferred_element_type=jnp.float32,
    )
    soft_ref[0, 0, :] = nd[0, :] / nd[1, :]
    sbit = jax.lax.shift_right_logical(
        jax.lax.bitcast_convert_type(t, jnp.uint32), jnp.uint32(31)
    )
    key = jax.lax.bitcast_convert_type(base_ref[0] | sbit, jnp.float32)
    packed = jax.lax.bitcast_convert_type(
        jnp.min(jnp.where(d == dmin, key, jnp.float32(2.0)), axis=0),
        jnp.uint32,
    )
    idx_ref[0, 0, :] = ((packed >> jnp.uint32(1)) & jnp.uint32(0x3FF)).astype(
        jnp.int32
    )
    sign = 1.0 - 2.0 * (packed & jnp.uint32(1)).astype(jnp.float32)
    hard_ref[0, 0, :] = x + sign * dmin[0, :]


@functools.partial(jax.jit, static_argnames=("interpret",))
def _run(z, codes, interpret=False):
    B, C, H, W = z.shape
    K = codes.shape[1]
    S = B * H * W
    SBLK = 2304
    xs = z.reshape(B, C, H * W).transpose(1, 0, 2).reshape(C, 1, S)
    wcol = codes.reshape(C, K, 1)
    lhs = jnp.stack([codes, jnp.ones_like(codes)], axis=1)
    base = (
        (jnp.arange(K, dtype=jnp.uint32) << jnp.uint32(1))
        | jnp.uint32(0x3F800000)
    ).reshape(1, K, 1)
    out_shape = [
        jax.ShapeDtypeStruct((C, 1, S), jnp.float32),
        jax.ShapeDtypeStruct((C, 1, S), jnp.float32),
        jax.ShapeDtypeStruct((C, 1, S), jnp.int32),
    ]
    grid = (C, S // SBLK)
    soft, hard, idx = pl.pallas_call(
        _vq_tile,
        grid=grid,
        in_specs=[
            pl.BlockSpec((1, 1, SBLK), lambda c, s: (c, 0, s)),
            pl.BlockSpec((1, K, 1), lambda c, s: (c, 0, 0)),
            pl.BlockSpec((1, 2, K), lambda c, s: (c, 0, 0)),
            pl.BlockSpec((1, K, 1), lambda c, s: (0, 0, 0)),
        ],
        out_specs=[
            pl.BlockSpec((1, 1, SBLK), lambda c, s: (c, 0, s)),
            pl.BlockSpec((1, 1, SBLK), lambda c, s: (c, 0, s)),
            pl.BlockSpec((1, 1, SBLK), lambda c, s: (c, 0, s)),
        ],
        out_shape=out_shape,
        compiler_params=pltpu.CompilerParams(
            dimension_semantics=("parallel", "parallel"),
        ),
        interpret=interpret,
    )(xs, wcol, lhs, base)

    def back(a):
        return a.reshape(C, B, H, W).transpose(1, 2, 3, 0)

    return back(soft), back(hard), back(idx)


def kernel(z, codes):
    return _run(z, codes)
